# baseline (speedup 1.0000x reference)
import functools

import jax
import jax.numpy as jnp
import numpy as np
from jax import lax
from jax.experimental import pallas as pl
from jax.experimental.pallas import tpu as pltpu
from jax.experimental.pallas import tpu_sc as plsc

NF = 100
NN = 120
NSEG = 128
B = 16384
D = 128
NCORES = 2
NSUB = 16
NW = NCORES * NSUB
ROWS = B // NW
CH = 128
NCHUNK = ROWS // CH


ZROWS = NSEG // NSUB


def _sc_segsum(reps_hbm, tgt_hbm, sums_out,
               rows_v, tgt_v, zero_v, shared_sums, sems):
    cid = lax.axis_index("c")
    sid = lax.axis_index("s")
    wid = cid * NSUB + sid
    base = wid * ROWS

    for j in range(NCHUNK):
        pltpu.sync_copy(tgt_hbm.at[pl.ds(base + j * CH, CH)], tgt_v.at[j])
    cps = [
        pltpu.async_copy(reps_hbm.at[pl.ds(base + j * CH, CH)],
                         rows_v.at[pl.ds(j * CH, CH)], sems[j])
        for j in range(NCHUNK)
    ]

    zeros16 = jnp.zeros((16,), jnp.float32)
    for r in range(ZROWS):
        for l in range(D // 16):
            zero_v[r, pl.ds(l * 16, 16)] = zeros16
    pltpu.sync_copy(zero_v, shared_sums.at[pl.ds(sid * ZROWS, ZROWS)])

    plsc.subcore_barrier()

    for j in range(NCHUNK):
        cps[j].wait()
        pltpu.sync_copy(rows_v.at[pl.ds(j * CH, CH)],
                        shared_sums.at[tgt_v.at[j]], add=True)

    plsc.subcore_barrier()

    @pl.when(sid == 0)
    def _writeout():
        pltpu.sync_copy(shared_sums, sums_out.at[cid])


@functools.lru_cache(maxsize=1)
def _make_seg_call():
    return pl.kernel(
        _sc_segsum,
        out_type=jax.ShapeDtypeStruct((NCORES, NSEG, D), jnp.float32),
        mesh=plsc.VectorSubcoreMesh(core_axis_name="c", subcore_axis_name="s"),
        scratch_types=[
            pltpu.VMEM((ROWS, D), jnp.float32),
            pltpu.VMEM((NCHUNK, CH), jnp.int32),
            pltpu.VMEM((ZROWS, D), jnp.float32),
            pltpu.VMEM_SHARED((NSEG, D), jnp.float32),
            [pltpu.SemaphoreType.DMA] * NCHUNK,
        ],
    )


def _tc_counts(tgt_ref, o_ref):
    f32 = jnp.float32
    classrow = lax.broadcasted_iota(jnp.int32, (1, NSEG), 1)
    acc = jnp.zeros((1, NSEG), f32)
    for k in range(NW * NCHUNK):
        col = tgt_ref[:, k:k + 1]
        oh = jnp.where(col == classrow, 1.0, 0.0)
        acc = acc + jnp.sum(oh, axis=0, keepdims=True)
    o_ref[...] = jnp.broadcast_to(acc, (8, NSEG))


_counts_call = pl.pallas_call(
    _tc_counts,
    out_shape=jax.ShapeDtypeStruct((8, NSEG), jnp.float32),
)


def _tc_tail(s_ref, c_ref, lm_ref, td_ref, o_ref):
    f32 = jnp.float32
    sums = s_ref[0] + s_ref[1]
    cnt_row = c_ref[0:1, :]

    lm = lm_ref[0:1, :]
    ri = lax.broadcasted_iota(jnp.int32, (NSEG, NSEG), 0)
    ci = lax.broadcasted_iota(jnp.int32, (NSEG, NSEG), 1)
    eye = ri == ci

    a_fine = jnp.where(eye & (ri < NF), 1.0, 0.0)
    a_coarse = jnp.where((ri >= NF) & (ri < NN) & (lm == ri), 1.0, 0.0)
    amat = (a_fine + a_coarse).astype(f32)

    node_sums = jnp.dot(amat, sums, preferred_element_type=f32)
    node_cnt = jnp.sum(amat * cnt_row, axis=1, keepdims=True)

    present = node_cnt > 0.0
    safe = jnp.where(present, node_cnt, 1.0)
    means = node_sums / safe

    gram = lax.dot_general(means, means, (((1,), (1,)), ((), ())),
                           preferred_element_type=f32)
    diag_col = jnp.sum(jnp.where(eye, gram, 0.0), axis=1, keepdims=True)
    diag_row = jnp.sum(jnp.where(eye, gram, 0.0), axis=0, keepdims=True)
    d2 = jnp.maximum(diag_col + diag_row - 2.0 * gram, 0.0)
    dist = jnp.sqrt(d2)

    node_cnt_row = jnp.sum(jnp.where(eye, node_cnt * jnp.ones((1, NSEG), f32),
                                     0.0), axis=0, keepdims=True)
    w = jnp.where((node_cnt > 0.0) & (node_cnt_row > 0.0)
                  & (ri < ci) & (ci < NN), 1.0, 0.0).astype(f32)

    y = td_ref[...]
    m = jnp.sum(w)
    mx = jnp.sum(w * dist) / m
    my = jnp.sum(w * y) / m
    xc = w * (dist - mx)
    yc = w * (y - my)
    num = jnp.sum(xc * yc)
    den = jnp.sqrt(jnp.sum(xc * xc)) * jnp.sqrt(jnp.sum(yc * yc))
    res = 1.0 - num / den
    res = jnp.where(jnp.isnan(res), jnp.array(1.0, f32), res)
    o_ref[0, 0] = res


_tail_call = pl.pallas_call(
    _tc_tail,
    out_shape=jax.ShapeDtypeStruct((1, 1), jnp.float32),
    out_specs=pl.BlockSpec(memory_space=pltpu.SMEM),
)


def kernel(representations, targets_fine, label_map, tree_dist):
    tgt2d = targets_fine.reshape(NW * NCHUNK, CH)
    lm_pad = jnp.pad(label_map[:, 1], (0, NSEG - NF))[None, :]
    td_pad = jnp.pad(tree_dist, ((0, NSEG - NN), (0, NSEG - NN)))

    sums2 = _make_seg_call()(representations, targets_fine)
    cnts = _counts_call(tgt2d)
    outb = _tail_call(sums2, cnts, lm_pad, td_pad)
    return outb.reshape(())


# speedup vs baseline: 1.0011x; 1.0011x over previous
import functools

import jax
import jax.numpy as jnp
import numpy as np
from jax import lax
from jax.experimental import pallas as pl
from jax.experimental.pallas import tpu as pltpu
from jax.experimental.pallas import tpu_sc as plsc

NF = 100
NN = 120
NSEG = 128
B = 16384
D = 128
NCORES = 2
NSUB = 16
NW = NCORES * NSUB
ROWS = B // NW
CH = 128
NCHUNK = ROWS // CH


def _sc_segsum(reps_hbm, tgt_hbm, zsum_hbm, sums_out,
               rows_v, tgt_v, shared_sums, sems):
    cid = lax.axis_index("c")
    sid = lax.axis_index("s")
    wid = cid * NSUB + sid
    base = wid * ROWS

    for j in range(NCHUNK):
        pltpu.sync_copy(tgt_hbm.at[pl.ds(base + j * CH, CH)], tgt_v.at[j])
    cps = [
        pltpu.async_copy(reps_hbm.at[pl.ds(base + j * CH, CH)],
                         rows_v.at[pl.ds(j * CH, CH)], sems[j])
        for j in range(NCHUNK)
    ]

    @pl.when(sid == 0)
    def _zero():
        pltpu.sync_copy(zsum_hbm, shared_sums)

    plsc.subcore_barrier()

    for j in range(NCHUNK):
        cps[j].wait()
        pltpu.sync_copy(rows_v.at[pl.ds(j * CH, CH)],
                        shared_sums.at[tgt_v.at[j]], add=True)

    plsc.subcore_barrier()

    @pl.when(sid == 0)
    def _writeout():
        pltpu.sync_copy(shared_sums, sums_out.at[cid])


@functools.lru_cache(maxsize=1)
def _make_seg_call():
    return pl.kernel(
        _sc_segsum,
        out_type=jax.ShapeDtypeStruct((NCORES, NSEG, D), jnp.float32),
        mesh=plsc.VectorSubcoreMesh(core_axis_name="c", subcore_axis_name="s"),
        scratch_types=[
            pltpu.VMEM((ROWS, D), jnp.float32),
            pltpu.VMEM((NCHUNK, CH), jnp.int32),
            pltpu.VMEM_SHARED((NSEG, D), jnp.float32),
            [pltpu.SemaphoreType.DMA] * NCHUNK,
        ],
    )


def _tc_counts(tgt_ref, o_ref):
    f32 = jnp.float32
    classrow = lax.broadcasted_iota(jnp.int32, (1, NSEG), 1)
    acc = jnp.zeros((1, NSEG), f32)
    for k in range(NW * NCHUNK):
        col = tgt_ref[:, k:k + 1]
        oh = jnp.where(col == classrow, 1.0, 0.0)
        acc = acc + jnp.sum(oh, axis=0, keepdims=True)
    o_ref[...] = jnp.broadcast_to(acc, (8, NSEG))


_counts_call = pl.pallas_call(
    _tc_counts,
    out_shape=jax.ShapeDtypeStruct((8, NSEG), jnp.float32),
)


def _tc_tail(s_ref, c_ref, lm_ref, td_ref, o_ref):
    f32 = jnp.float32
    sums = s_ref[0] + s_ref[1]
    cnt_row = c_ref[0:1, :]

    lm = lm_ref[0:1, :]
    ri = lax.broadcasted_iota(jnp.int32, (NSEG, NSEG), 0)
    ci = lax.broadcasted_iota(jnp.int32, (NSEG, NSEG), 1)
    eye = ri == ci

    a_fine = jnp.where(eye & (ri < NF), 1.0, 0.0)
    a_coarse = jnp.where((ri >= NF) & (ri < NN) & (lm == ri), 1.0, 0.0)
    amat = (a_fine + a_coarse).astype(f32)

    node_sums = jnp.dot(amat, sums, preferred_element_type=f32)
    node_cnt = jnp.sum(amat * cnt_row, axis=1, keepdims=True)

    present = node_cnt > 0.0
    safe = jnp.where(present, node_cnt, 1.0)
    means = node_sums / safe

    gram = lax.dot_general(means, means, (((1,), (1,)), ((), ())),
                           preferred_element_type=f32)
    diag_col = jnp.sum(jnp.where(eye, gram, 0.0), axis=1, keepdims=True)
    diag_row = jnp.sum(jnp.where(eye, gram, 0.0), axis=0, keepdims=True)
    d2 = jnp.maximum(diag_col + diag_row - 2.0 * gram, 0.0)
    dist = jnp.sqrt(d2)

    node_cnt_row = jnp.sum(jnp.where(eye, node_cnt * jnp.ones((1, NSEG), f32),
                                     0.0), axis=0, keepdims=True)
    w = jnp.where((node_cnt > 0.0) & (node_cnt_row > 0.0)
                  & (ri < ci) & (ci < NN), 1.0, 0.0).astype(f32)

    y = td_ref[...]
    m = jnp.sum(w)
    mx = jnp.sum(w * dist) / m
    my = jnp.sum(w * y) / m
    xc = w * (dist - mx)
    yc = w * (y - my)
    num = jnp.sum(xc * yc)
    den = jnp.sqrt(jnp.sum(xc * xc)) * jnp.sqrt(jnp.sum(yc * yc))
    res = 1.0 - num / den
    res = jnp.where(jnp.isnan(res), jnp.array(1.0, f32), res)
    o_ref[0, 0] = res


_tail_call = pl.pallas_call(
    _tc_tail,
    out_shape=jax.ShapeDtypeStruct((1, 1), jnp.float32),
    out_specs=pl.BlockSpec(memory_space=pltpu.SMEM),
)


def kernel(representations, targets_fine, label_map, tree_dist):
    tgt2d = targets_fine.reshape(NW * NCHUNK, CH)
    lm_pad = jnp.pad(label_map[:, 1], (0, NSEG - NF))[None, :]
    td_pad = jnp.pad(tree_dist, ((0, NSEG - NN), (0, NSEG - NN)))

    zsum = np.zeros((NSEG, D), np.float32)
    sums2 = _make_seg_call()(representations, targets_fine, zsum)
    cnts = _counts_call(tgt2d)
    outb = _tail_call(sums2, cnts, lm_pad, td_pad)
    return outb.reshape(())


# speedup vs baseline: 1.0491x; 1.0480x over previous
import functools

import jax
import jax.numpy as jnp
import numpy as np
from jax import lax
from jax.experimental import pallas as pl
from jax.experimental.pallas import tpu as pltpu
from jax.experimental.pallas import tpu_sc as plsc

NF = 100
NN = 120
NSEG = 128
B = 16384
D = 128
NCORES = 2
NSUB = 16
NW = NCORES * NSUB
ROWS = B // NW
CH = 128
NCHUNK = ROWS // CH


def _sc_segsum(reps_hbm, tgt_hbm, zsum_hbm, sums_out,
               rows_v, tgt_v, shared_sums, sems):
    cid = lax.axis_index("c")
    sid = lax.axis_index("s")
    wid = cid * NSUB + sid
    base = wid * ROWS

    pltpu.sync_copy(tgt_hbm.at[pl.ds(wid * NCHUNK, NCHUNK)], tgt_v)
    cps = [
        pltpu.async_copy(reps_hbm.at[pl.ds(base + j * CH, CH)],
                         rows_v.at[pl.ds(j * CH, CH)], sems[j])
        for j in range(NCHUNK)
    ]

    @pl.when(sid == 0)
    def _zero():
        pltpu.sync_copy(zsum_hbm, shared_sums)

    plsc.subcore_barrier()

    for j in range(NCHUNK):
        cps[j].wait()
        pltpu.sync_copy(rows_v.at[pl.ds(j * CH, CH)],
                        shared_sums.at[tgt_v.at[j]], add=True)

    plsc.subcore_barrier()

    @pl.when(sid == 0)
    def _writeout():
        pltpu.sync_copy(shared_sums, sums_out.at[cid])


@functools.lru_cache(maxsize=1)
def _make_seg_call():
    return pl.kernel(
        _sc_segsum,
        out_type=jax.ShapeDtypeStruct((NCORES, NSEG, D), jnp.float32),
        mesh=plsc.VectorSubcoreMesh(core_axis_name="c", subcore_axis_name="s"),
        scratch_types=[
            pltpu.VMEM((ROWS, D), jnp.float32),
            pltpu.VMEM((NCHUNK, CH), jnp.int32),
            pltpu.VMEM_SHARED((NSEG, D), jnp.float32),
            [pltpu.SemaphoreType.DMA] * NCHUNK,
        ],
    )


def _tc_counts(tgt_ref, o_ref):
    f32 = jnp.float32
    classrow = lax.broadcasted_iota(jnp.int32, (1, NSEG), 1)
    acc = jnp.zeros((1, NSEG), f32)
    for k in range(NW * NCHUNK):
        col = tgt_ref[:, k:k + 1]
        oh = jnp.where(col == classrow, 1.0, 0.0)
        acc = acc + jnp.sum(oh, axis=0, keepdims=True)
    o_ref[...] = jnp.broadcast_to(acc, (8, NSEG))


_counts_call = pl.pallas_call(
    _tc_counts,
    out_shape=jax.ShapeDtypeStruct((8, NSEG), jnp.float32),
)


def _tc_tail(s_ref, c_ref, lm_ref, td_ref, o_ref):
    f32 = jnp.float32
    sums = s_ref[0] + s_ref[1]
    cnt_row = c_ref[0:1, :]

    lm = lm_ref[0:1, :]
    ri = lax.broadcasted_iota(jnp.int32, (NSEG, NSEG), 0)
    ci = lax.broadcasted_iota(jnp.int32, (NSEG, NSEG), 1)
    eye = ri == ci

    a_fine = jnp.where(eye & (ri < NF), 1.0, 0.0)
    a_coarse = jnp.where((ri >= NF) & (ri < NN) & (lm == ri), 1.0, 0.0)
    amat = (a_fine + a_coarse).astype(f32)

    node_sums = jnp.dot(amat, sums, preferred_element_type=f32)
    node_cnt = jnp.sum(amat * cnt_row, axis=1, keepdims=True)

    present = node_cnt > 0.0
    safe = jnp.where(present, node_cnt, 1.0)
    means = node_sums / safe

    gram = lax.dot_general(means, means, (((1,), (1,)), ((), ())),
                           preferred_element_type=f32)
    diag_col = jnp.sum(jnp.where(eye, gram, 0.0), axis=1, keepdims=True)
    diag_row = jnp.sum(jnp.where(eye, gram, 0.0), axis=0, keepdims=True)
    d2 = jnp.maximum(diag_col + diag_row - 2.0 * gram, 0.0)
    dist = jnp.sqrt(d2)

    node_cnt_row = jnp.sum(jnp.where(eye, node_cnt * jnp.ones((1, NSEG), f32),
                                     0.0), axis=0, keepdims=True)
    w = jnp.where((node_cnt > 0.0) & (node_cnt_row > 0.0)
                  & (ri < ci) & (ci < NN), 1.0, 0.0).astype(f32)

    y = td_ref[...]
    m = jnp.sum(w)
    mx = jnp.sum(w * dist) / m
    my = jnp.sum(w * y) / m
    xc = w * (dist - mx)
    yc = w * (y - my)
    num = jnp.sum(xc * yc)
    den = jnp.sqrt(jnp.sum(xc * xc)) * jnp.sqrt(jnp.sum(yc * yc))
    res = 1.0 - num / den
    res = jnp.where(jnp.isnan(res), jnp.array(1.0, f32), res)
    o_ref[0, 0] = res


_tail_call = pl.pallas_call(
    _tc_tail,
    out_shape=jax.ShapeDtypeStruct((1, 1), jnp.float32),
    out_specs=pl.BlockSpec(memory_space=pltpu.SMEM),
)


def kernel(representations, targets_fine, label_map, tree_dist):
    tgt2d = targets_fine.reshape(NW * NCHUNK, CH)
    lm_pad = jnp.pad(label_map[:, 1], (0, NSEG - NF))[None, :]
    td_pad = jnp.pad(tree_dist, ((0, NSEG - NN), (0, NSEG - NN)))

    zsum = np.zeros((NSEG, D), np.float32)
    sums2 = _make_seg_call()(representations, tgt2d, zsum)
    cnts = _counts_call(tgt2d)
    outb = _tail_call(sums2, cnts, lm_pad, td_pad)
    return outb.reshape(())


# speedup vs baseline: 1.0514x; 1.0021x over previous
import functools

import jax
import jax.numpy as jnp
import numpy as np
from jax import lax
from jax.experimental import pallas as pl
from jax.experimental.pallas import tpu as pltpu
from jax.experimental.pallas import tpu_sc as plsc

NF = 100
NN = 120
NSEG = 128
B = 16384
D = 128
NCORES = 2
NSUB = 16
NW = NCORES * NSUB
ROWS = B // NW
CH = 128
NCHUNK = ROWS // CH


def _sc_segsum(reps_hbm, tgt_hbm, zsum_hbm, sums_out,
               rows_v, tgt_v, shared_sums, sems, ssems):
    cid = lax.axis_index("c")
    sid = lax.axis_index("s")
    wid = cid * NSUB + sid
    base = wid * ROWS

    pltpu.sync_copy(tgt_hbm.at[pl.ds(wid * NCHUNK, NCHUNK)], tgt_v)
    cps = [
        pltpu.async_copy(reps_hbm.at[pl.ds(base + j * CH, CH)],
                         rows_v.at[pl.ds(j * CH, CH)], sems[j])
        for j in range(NCHUNK)
    ]

    @pl.when(sid == 0)
    def _zero():
        pltpu.sync_copy(zsum_hbm, shared_sums)

    plsc.subcore_barrier()

    scs = []
    for j in range(NCHUNK):
        cps[j].wait()
        scs.append(pltpu.async_copy(rows_v.at[pl.ds(j * CH, CH)],
                                    shared_sums.at[tgt_v.at[j]], ssems[j],
                                    add=True))
    for j in range(NCHUNK):
        scs[j].wait()

    plsc.subcore_barrier()

    @pl.when(sid == 0)
    def _writeout():
        pltpu.sync_copy(shared_sums, sums_out.at[cid])


@functools.lru_cache(maxsize=1)
def _make_seg_call():
    return pl.kernel(
        _sc_segsum,
        out_type=jax.ShapeDtypeStruct((NCORES, NSEG, D), jnp.float32),
        mesh=plsc.VectorSubcoreMesh(core_axis_name="c", subcore_axis_name="s"),
        scratch_types=[
            pltpu.VMEM((ROWS, D), jnp.float32),
            pltpu.VMEM((NCHUNK, CH), jnp.int32),
            pltpu.VMEM_SHARED((NSEG, D), jnp.float32),
            [pltpu.SemaphoreType.DMA] * NCHUNK,
            [pltpu.SemaphoreType.DMA] * NCHUNK,
        ],
    )


def _tc_counts(tgt_ref, o_ref):
    f32 = jnp.float32
    classrow = lax.broadcasted_iota(jnp.int32, (1, NSEG), 1)
    acc = jnp.zeros((1, NSEG), f32)
    for k in range(NW * NCHUNK):
        col = tgt_ref[:, k:k + 1]
        oh = jnp.where(col == classrow, 1.0, 0.0)
        acc = acc + jnp.sum(oh, axis=0, keepdims=True)
    o_ref[...] = jnp.broadcast_to(acc, (8, NSEG))


_counts_call = pl.pallas_call(
    _tc_counts,
    out_shape=jax.ShapeDtypeStruct((8, NSEG), jnp.float32),
)


def _tc_tail(s_ref, c_ref, lm_ref, td_ref, o_ref):
    f32 = jnp.float32
    sums = s_ref[0] + s_ref[1]
    cnt_row = c_ref[0:1, :]

    lm = lm_ref[0:1, :]
    ri = lax.broadcasted_iota(jnp.int32, (NSEG, NSEG), 0)
    ci = lax.broadcasted_iota(jnp.int32, (NSEG, NSEG), 1)
    eye = ri == ci

    a_fine = jnp.where(eye & (ri < NF), 1.0, 0.0)
    a_coarse = jnp.where((ri >= NF) & (ri < NN) & (lm == ri), 1.0, 0.0)
    amat = (a_fine + a_coarse).astype(f32)

    node_sums = jnp.dot(amat, sums, preferred_element_type=f32)
    node_cnt = jnp.sum(amat * cnt_row, axis=1, keepdims=True)

    present = node_cnt > 0.0
    safe = jnp.where(present, node_cnt, 1.0)
    means = node_sums / safe

    gram = lax.dot_general(means, means, (((1,), (1,)), ((), ())),
                           preferred_element_type=f32)
    diag_col = jnp.sum(jnp.where(eye, gram, 0.0), axis=1, keepdims=True)
    diag_row = jnp.sum(jnp.where(eye, gram, 0.0), axis=0, keepdims=True)
    d2 = jnp.maximum(diag_col + diag_row - 2.0 * gram, 0.0)
    dist = jnp.sqrt(d2)

    node_cnt_row = jnp.sum(jnp.where(eye, node_cnt * jnp.ones((1, NSEG), f32),
                                     0.0), axis=0, keepdims=True)
    w = jnp.where((node_cnt > 0.0) & (node_cnt_row > 0.0)
                  & (ri < ci) & (ci < NN), 1.0, 0.0).astype(f32)

    y = td_ref[...]
    m = jnp.sum(w)
    mx = jnp.sum(w * dist) / m
    my = jnp.sum(w * y) / m
    xc = w * (dist - mx)
    yc = w * (y - my)
    num = jnp.sum(xc * yc)
    den = jnp.sqrt(jnp.sum(xc * xc)) * jnp.sqrt(jnp.sum(yc * yc))
    res = 1.0 - num / den
    res = jnp.where(jnp.isnan(res), jnp.array(1.0, f32), res)
    o_ref[0, 0] = res


_tail_call = pl.pallas_call(
    _tc_tail,
    out_shape=jax.ShapeDtypeStruct((1, 1), jnp.float32),
    out_specs=pl.BlockSpec(memory_space=pltpu.SMEM),
)


def kernel(representations, targets_fine, label_map, tree_dist):
    tgt2d = targets_fine.reshape(NW * NCHUNK, CH)
    lm_pad = jnp.pad(label_map[:, 1], (0, NSEG - NF))[None, :]
    td_pad = jnp.pad(tree_dist, ((0, NSEG - NN), (0, NSEG - NN)))

    zsum = np.zeros((NSEG, D), np.float32)
    sums2 = _make_seg_call()(representations, tgt2d, zsum)
    cnts = _counts_call(tgt2d)
    outb = _tail_call(sums2, cnts, lm_pad, td_pad)
    return outb.reshape(())


# speedup vs baseline: 1.0543x; 1.0028x over previous
import functools

import jax
import jax.numpy as jnp
import numpy as np
from jax import lax
from jax.experimental import pallas as pl
from jax.experimental.pallas import tpu as pltpu
from jax.experimental.pallas import tpu_sc as plsc

NF = 100
NN = 120
NSEG = 128
B = 16384
D = 128
NCORES = 2
NSUB = 16
NW = NCORES * NSUB
ROWS = B // NW
CH = 128
NCHUNK = ROWS // CH


def _sc_segsum(reps_hbm, tgt_hbm, zsum_hbm, sums_out,
               rows_v, tgt_v, shared_sums, sems, ssems):
    cid = lax.axis_index("c")
    sid = lax.axis_index("s")
    wid = cid * NSUB + sid
    base = wid * ROWS

    pltpu.sync_copy(tgt_hbm.at[pl.ds(wid * NCHUNK, NCHUNK)], tgt_v)
    cps = [
        pltpu.async_copy(reps_hbm.at[pl.ds(base + j * CH, CH)],
                         rows_v.at[pl.ds(j * CH, CH)], sems[j])
        for j in range(NCHUNK)
    ]

    @pl.when(sid == 0)
    def _zero():
        pltpu.sync_copy(zsum_hbm, shared_sums)

    plsc.subcore_barrier()

    scs = []
    for j in range(NCHUNK):
        cps[j].wait()
        scs.append(pltpu.async_copy(rows_v.at[pl.ds(j * CH, CH)],
                                    shared_sums.at[tgt_v.at[j]], ssems[j],
                                    add=True))
    for j in range(NCHUNK):
        scs[j].wait()

    plsc.subcore_barrier()

    @pl.when(sid == 0)
    def _writeout():
        pltpu.sync_copy(shared_sums, sums_out.at[cid])


@functools.lru_cache(maxsize=1)
def _make_seg_call():
    return pl.kernel(
        _sc_segsum,
        out_type=jax.ShapeDtypeStruct((NCORES, NSEG, D), jnp.float32),
        mesh=plsc.VectorSubcoreMesh(core_axis_name="c", subcore_axis_name="s"),
        scratch_types=[
            pltpu.VMEM((ROWS, D), jnp.float32),
            pltpu.VMEM((NCHUNK, CH), jnp.int32),
            pltpu.VMEM_SHARED((NSEG, D), jnp.float32),
            [pltpu.SemaphoreType.DMA] * NCHUNK,
            [pltpu.SemaphoreType.DMA] * NCHUNK,
        ],
    )


def _tc_counts(tgt_ref, o_ref):
    f32 = jnp.float32
    classrow = lax.broadcasted_iota(jnp.int32, (1, NSEG), 1)
    acc = jnp.zeros((1, NSEG), f32)
    for k in range(NW * NCHUNK):
        col = tgt_ref[:, k:k + 1]
        oh = jnp.where(col == classrow, 1.0, 0.0)
        acc = acc + jnp.sum(oh, axis=0, keepdims=True)
    o_ref[...] = jnp.broadcast_to(acc, (8, NSEG))


_counts_call = pl.pallas_call(
    _tc_counts,
    out_shape=jax.ShapeDtypeStruct((8, NSEG), jnp.float32),
)


def _tc_tail(s_ref, c_ref, lm_ref, td_ref, o_ref):
    f32 = jnp.float32
    sums = s_ref[0] + s_ref[1]
    cnt_row = c_ref[0:1, :]

    lm = lm_ref[0:1, :]
    ri = lax.broadcasted_iota(jnp.int32, (NSEG, NSEG), 0)
    ci = lax.broadcasted_iota(jnp.int32, (NSEG, NSEG), 1)
    eye = ri == ci

    amat = jnp.where((eye & (ri < NF))
                     | ((ri >= NF) & (ri < NN) & (lm == ri)),
                     1.0, 0.0).astype(f32)

    node_sums = jnp.dot(amat, sums, preferred_element_type=f32)
    node_cnt = jnp.sum(amat * cnt_row, axis=1, keepdims=True)

    present = node_cnt > 0.0
    safe = jnp.where(present, node_cnt, 1.0)
    means = node_sums / safe

    gram = lax.dot_general(means, means, (((1,), (1,)), ((), ())),
                           preferred_element_type=f32)
    diag_col = jnp.sum(means * means, axis=1, keepdims=True)
    diag_row = jnp.sum(jnp.where(eye, gram, 0.0), axis=0, keepdims=True)
    d2 = jnp.maximum(diag_col + diag_row - 2.0 * gram, 0.0)
    dist = jnp.sqrt(d2)

    node_cnt_row = lax.dot_general(cnt_row, amat, (((1,), (1,)), ((), ())),
                                   preferred_element_type=f32)
    w = jnp.where((node_cnt > 0.0) & (node_cnt_row > 0.0)
                  & (ri < ci) & (ci < NN), 1.0, 0.0).astype(f32)

    y = td_ref[...]
    m = jnp.sum(w)
    mx = jnp.sum(w * dist) / m
    my = jnp.sum(w * y) / m
    xc = w * (dist - mx)
    yc = w * (y - my)
    num = jnp.sum(xc * yc)
    den = jnp.sqrt(jnp.sum(xc * xc)) * jnp.sqrt(jnp.sum(yc * yc))
    res = 1.0 - num / den
    res = jnp.where(jnp.isnan(res), jnp.array(1.0, f32), res)
    o_ref[0, 0] = res


_tail_call = pl.pallas_call(
    _tc_tail,
    out_shape=jax.ShapeDtypeStruct((1, 1), jnp.float32),
    out_specs=pl.BlockSpec(memory_space=pltpu.SMEM),
)


def kernel(representations, targets_fine, label_map, tree_dist):
    tgt2d = targets_fine.reshape(NW * NCHUNK, CH)
    lm_pad = jnp.pad(label_map[:, 1], (0, NSEG - NF))[None, :]
    td_pad = jnp.pad(tree_dist, ((0, NSEG - NN), (0, NSEG - NN)))

    zsum = np.zeros((NSEG, D), np.float32)
    sums2 = _make_seg_call()(representations, tgt2d, zsum)
    cnts = _counts_call(tgt2d)
    outb = _tail_call(sums2, cnts, lm_pad, td_pad)
    return outb.reshape(())
